# baseline (speedup 1.0000x reference)
import functools

import jax
import jax.numpy as jnp
from jax import lax
from jax.experimental import pallas as pl
from jax.experimental.pallas import tpu as pltpu
from jax.experimental.pallas import tpu_sc as plsc

D = 2048
R = D // 4
L = 24
N = 576
T = 128
FINAL_K = 64

_PREC = lax.Precision.DEFAULT


def _dotT(a, w, precision=_PREC):
    return lax.dot_general(a.astype(jnp.bfloat16), w.astype(jnp.bfloat16),
                           (((1,), (1,)), ((), ())),
                           preferred_element_type=jnp.float32,
                           precision=precision)


def _pool_body(x_ref, y_ref):
    y_ref[0] = jnp.mean(x_ref[0], axis=0, keepdims=True)


def _recurrence_body(text_ref, y_ref, w1_ref, w1b_ref, wc_ref, wcb_ref,
                     wi_ref, wib_ref, wf_ref, wfb_ref, wq_ref, lnw_ref,
                     lnb_ref, q_ref):
    tmean = jnp.mean(text_ref[...], axis=0, keepdims=True)
    mu = jnp.mean(tmean, axis=1, keepdims=True)
    var = jnp.mean((tmean - mu) ** 2, axis=1, keepdims=True)
    tg = (tmean - mu) / jnp.sqrt(var + 1e-5)

    def step(l, c):
        yl = y_ref[pl.ds(l, 1), :]
        cn = jax.nn.sigmoid(c)
        comb = jnp.concatenate([cn, yl, tg], axis=1)
        s = jax.nn.relu(_dotT(comb, w1_ref[...]) + w1b_ref[...])
        ct = jnp.tanh(_dotT(s, wc_ref[...]) + wcb_ref[...])
        gi = jax.nn.sigmoid(_dotT(s, wi_ref[...]) + wib_ref[...])
        gf = jax.nn.sigmoid(_dotT(s, wf_ref[...]) + wfb_ref[...])
        return gf * c + gi * ct

    c = lax.fori_loop(0, L - 1, step, jnp.zeros((1, D), jnp.float32))
    qpre = _dotT(c, wq_ref[...])
    mu = jnp.mean(qpre, axis=1, keepdims=True)
    var = jnp.mean((qpre - mu) ** 2, axis=1, keepdims=True)
    q_ref[...] = ((qpre - mu) / jnp.sqrt(var + 1e-5)) * lnw_ref[...] + lnb_ref[...]


def _score_body(x_ref, wk_ref, wqc_ref, sb_ref, z_ref):
    x = x_ref[0].astype(jnp.bfloat16)
    wk = wk_ref[...].astype(jnp.bfloat16)
    vT = lax.dot_general(wk, x, (((1,), (1,)), ((), ())),
                         preferred_element_type=jnp.float32,
                         precision=_PREC)
    wqc = wqc_ref[...]
    dot = jnp.sum(vT * wqc, axis=0, keepdims=True)
    sumv = jnp.sum(vT, axis=0, keepdims=True)
    ssq = jnp.sum(vT * vT, axis=0, keepdims=True)
    mu = sumv * (1.0 / D)
    var = ssq * (1.0 / D) - mu * mu
    denom = jnp.sqrt(var + 1e-5)
    s_sum = sb_ref[0]
    bq = sb_ref[1]
    score = (dot - mu * s_sum) / denom + bq
    m = jnp.mean(score)
    sd = jnp.sqrt(jnp.mean((score - m) ** 2))
    z_ref[0] = (score - m) / (sd + 1e-6)


_NTILE = 16
_PERW = (L * N) // _NTILE
_NVREG = _PERW // 16
_NEG = float("-inf")


def _bc16(x, dtype=jnp.int32):
    return jnp.full((16,), x, dtype)


def _shuf(v, idx):
    return lax.gather(
        v, idx[:, None],
        dimension_numbers=lax.GatherDimensionNumbers(
            offset_dims=(), collapsed_slice_dims=(0,), start_index_map=(0,)),
        slice_sizes=(1,),
        mode=lax.GatherScatterMode.PROMISE_IN_BOUNDS)


def _allreduce(v, op):
    lane = lax.broadcasted_iota(jnp.int32, (16,), 0)
    for s in (1, 2, 4, 8):
        v = op(v, _shuf(v, jnp.bitwise_xor(lane, s)))
    return v


def _allmax(v):
    return _allreduce(v, jnp.maximum)


def _allmin(v):
    return _allreduce(v, jnp.minimum)


def _sc_topk_body(z_hbm, x_hbm, out_hbm, buf_v, lc_v, sh_v, mc_v, fi_v,
                  idx8_v, rows_v, sem):
    wid = lax.axis_index("s")
    lane = lax.broadcasted_iota(jnp.int32, (16,), 0)
    lane0 = lane == 0
    on_core = lax.axis_index("c") == 0

    @pl.when(on_core)
    def _phase_a():
        base = wid * _PERW
        pltpu.sync_copy(z_hbm.at[pl.ds(base, _PERW)], buf_v)

        def round_body(r, carry):
            def scan_body(j, mc):
                maxv, maxi = mc
                v = buf_v[pl.ds(j * 16, 16)]
                idx = lane + j * 16
                upd = v > maxv
                return (jnp.where(upd, v, maxv), jnp.where(upd, idx, maxi))

            maxv, maxi = lax.fori_loop(
                0, _NVREG, scan_body,
                (jnp.full((16,), _NEG, jnp.float32), jnp.zeros((16,), jnp.int32)))
            m = _allmax(maxv)
            p = _allmin(jnp.where(maxv == m, maxi, _PERW))
            plsc.store_scatter(lc_v, [_bc16(r)], plsc.bitcast(m, jnp.int32),
                               mask=lane0)
            plsc.store_scatter(lc_v, [_bc16(r) + FINAL_K], p + base,
                               mask=lane0)
            plsc.store_scatter(buf_v, [p],
                               jnp.full((16,), _NEG, jnp.float32), mask=lane0)
            return carry

        lax.fori_loop(0, FINAL_K, round_body, 0)
        pltpu.sync_copy(lc_v, sh_v.at[wid])

    plsc.subcore_barrier()

    @pl.when(on_core & (wid == 0))
    def _phase_b():
        pltpu.sync_copy(sh_v, mc_v)
        heads0 = plsc.bitcast(
            plsc.load_gather(mc_v, [lane, jnp.zeros((16,), jnp.int32)]),
            jnp.float32)
        ptr0 = jnp.zeros((16,), jnp.int32)

        def merge_body(r, carry):
            heads, ptr = carry
            m = _allmax(heads)
            l = _allmin(jnp.where(heads == m, lane, _NTILE))
            eq = lane == l
            pv = _shuf(ptr, l)
            iv = plsc.load_gather(mc_v, [l, pv + FINAL_K])
            plsc.store_scatter(fi_v, [_bc16(r)], iv, mask=lane0)
            nxt = jnp.minimum(pv + 1, FINAL_K - 1)
            newhead = plsc.bitcast(plsc.load_gather(mc_v, [l, nxt]),
                                   jnp.float32)
            heads = jnp.where(eq & (pv + 1 >= FINAL_K), _NEG,
                              jnp.where(eq, newhead, heads))
            ptr = jnp.where(eq, ptr + 1, ptr)
            return (heads, ptr)

        lax.fori_loop(0, FINAL_K, merge_body, (heads0, ptr0))
        pltpu.sync_copy(fi_v, sh_v.at[0])

    plsc.subcore_barrier()

    @pl.when(on_core & (wid < 8))
    def _phase_c():
        pltpu.sync_copy(sh_v.at[0, pl.ds(wid * 8, 8)], idx8_v)
        pltpu.async_copy(x_hbm.at[idx8_v], rows_v, sem).wait()
        pltpu.sync_copy(rows_v, out_hbm.at[pl.ds(wid * 8, 8)])


def _sc_topk_gather(z_flat, x_flat):
    mesh = plsc.VectorSubcoreMesh(core_axis_name="c", subcore_axis_name="s")
    f = functools.partial(
        pl.kernel,
        out_type=jax.ShapeDtypeStruct((FINAL_K, D), jnp.float32),
        mesh=mesh,
        compiler_params=pltpu.CompilerParams(needs_layout_passes=False),
        scratch_types=[
            pltpu.VMEM((_PERW,), jnp.float32),
            pltpu.VMEM((2 * FINAL_K,), jnp.int32),
            pltpu.VMEM_SHARED((_NTILE, 2 * FINAL_K), jnp.int32),
            pltpu.VMEM((_NTILE, 2 * FINAL_K), jnp.int32),
            pltpu.VMEM((2 * FINAL_K,), jnp.int32),
            pltpu.VMEM((8,), jnp.int32),
            pltpu.VMEM((8, D), jnp.float32),
            pltpu.SemaphoreType.DMA,
        ],
    )(_sc_topk_body)
    return f(z_flat, x_flat)


def _ln_ref(x, w=None, b=None, eps=1e-5):
    mu = jnp.mean(x, axis=-1, keepdims=True)
    var = jnp.mean((x - mu) ** 2, axis=-1, keepdims=True)
    y = (x - mu) / jnp.sqrt(var + eps)
    if w is not None:
        y = y * w + b
    return y


def kernel(text_features, projected_layer_features, W1_w, W1_b, Wc_w, Wc_b,
           Wi_w, Wi_b, Wf_w, Wf_b, bc, bi, bf, Wq, Wk, ln_w, ln_b):
    X = projected_layer_features

    text_global = _ln_ref(jnp.mean(text_features, axis=0))
    y = jnp.mean(X, axis=1)

    def step(c_prev, y_l):
        c_prev_norm = jax.nn.sigmoid(c_prev)
        combined = jnp.concatenate([c_prev_norm, y_l, text_global], axis=-1)
        s = jax.nn.relu(combined @ W1_w.T + W1_b)
        c_tilde = jnp.tanh(s @ Wc_w.T + Wc_b + bc)
        i = jax.nn.sigmoid(s @ Wi_w.T + Wi_b + bi)
        f = jax.nn.sigmoid(s @ Wf_w.T + Wf_b + bf)
        c_l = f * c_prev + i * c_tilde
        return c_l, c_l

    c0 = jnp.zeros((D,), dtype=text_features.dtype)
    _, contexts = jax.lax.scan(step, c0, y)
    q = _ln_ref(contexts[-2] @ Wq.T, ln_w, ln_b)

    wq = (ln_w * q)
    sb = jnp.stack([jnp.sum(wq), jnp.dot(ln_b, q)])

    z = pl.pallas_call(
        _score_body,
        grid=(L,),
        in_specs=[
            pl.BlockSpec((1, N, D), lambda l: (l, 0, 0)),
            pl.BlockSpec((D, D), lambda l: (0, 0)),
            pl.BlockSpec((D, 1), lambda l: (0, 0)),
            pl.BlockSpec(memory_space=pltpu.SMEM),
        ],
        out_specs=pl.BlockSpec((1, 1, N), lambda l: (l, 0, 0)),
        out_shape=jax.ShapeDtypeStruct((L, 1, N), jnp.float32),
    )(X, Wk, wq.reshape(D, 1), sb)

    return _sc_topk_gather(z.reshape(-1), X.reshape(-1, D))


# speedup vs baseline: 1.0007x; 1.0007x over previous
import functools

import jax
import jax.numpy as jnp
from jax import lax
from jax.experimental import pallas as pl
from jax.experimental.pallas import tpu as pltpu
from jax.experimental.pallas import tpu_sc as plsc

D = 2048
R = D // 4
L = 24
N = 576
T = 128
FINAL_K = 64

_PREC = lax.Precision.DEFAULT


def _score_body(x_ref, wk_ref, wqc_ref, sb_ref, z_ref):
    x = x_ref[0].astype(jnp.bfloat16)
    wk = wk_ref[...].astype(jnp.bfloat16)
    vT = lax.dot_general(wk, x, (((1,), (1,)), ((), ())),
                         preferred_element_type=jnp.float32,
                         precision=_PREC)
    wqc = wqc_ref[...]
    dot = jnp.sum(vT * wqc, axis=0, keepdims=True)
    sumv = jnp.sum(vT, axis=0, keepdims=True)
    ssq = jnp.sum(vT * vT, axis=0, keepdims=True)
    mu = sumv * (1.0 / D)
    var = ssq * (1.0 / D) - mu * mu
    denom = jnp.sqrt(var + 1e-5)
    s_sum = sb_ref[0]
    bq = sb_ref[1]
    score = (dot - mu * s_sum) / denom + bq
    m = jnp.mean(score)
    sd = jnp.sqrt(jnp.mean((score - m) ** 2))
    z_ref[0] = (score - m) / (sd + 1e-6)


_NTILE = 16
_PERW = (L * N) // _NTILE
_NVREG = _PERW // 16
_NEG = float("-inf")


def _bc16(x, dtype=jnp.int32):
    return jnp.full((16,), x, dtype)


def _shuf(v, idx):
    return lax.gather(
        v, idx[:, None],
        dimension_numbers=lax.GatherDimensionNumbers(
            offset_dims=(), collapsed_slice_dims=(0,), start_index_map=(0,)),
        slice_sizes=(1,),
        mode=lax.GatherScatterMode.PROMISE_IN_BOUNDS)


def _allreduce(v, op):
    lane = lax.broadcasted_iota(jnp.int32, (16,), 0)
    for s in (1, 2, 4, 8):
        v = op(v, _shuf(v, jnp.bitwise_xor(lane, s)))
    return v


def _allmax(v):
    return _allreduce(v, jnp.maximum)


def _allmin(v):
    return _allreduce(v, jnp.minimum)


def _sc_topk_body(z_hbm, x_hbm, out_hbm, buf_v, lc_v, sh_v, mc_v, fi_v,
                  idx8_v, rows_v, sem):
    wid = lax.axis_index("s")
    lane = lax.broadcasted_iota(jnp.int32, (16,), 0)
    lane0 = lane == 0
    on_core = lax.axis_index("c") == 0

    @pl.when(on_core)
    def _phase_a():
        base = wid * _PERW
        pltpu.sync_copy(z_hbm.at[pl.ds(base, _PERW)], buf_v)

        def round_body(r, carry):
            def scan_body(j, mc):
                maxv, maxi = mc
                v = buf_v[pl.ds(j * 16, 16)]
                idx = lane + j * 16
                upd = v > maxv
                return (jnp.where(upd, v, maxv), jnp.where(upd, idx, maxi))

            maxv, maxi = lax.fori_loop(
                0, _NVREG, scan_body,
                (jnp.full((16,), _NEG, jnp.float32), jnp.zeros((16,), jnp.int32)))
            m = _allmax(maxv)
            p = _allmin(jnp.where(maxv == m, maxi, _PERW))
            plsc.store_scatter(lc_v, [_bc16(r)], plsc.bitcast(m, jnp.int32),
                               mask=lane0)
            plsc.store_scatter(lc_v, [_bc16(r) + FINAL_K], p + base,
                               mask=lane0)
            plsc.store_scatter(buf_v, [p],
                               jnp.full((16,), _NEG, jnp.float32), mask=lane0)
            return carry

        lax.fori_loop(0, FINAL_K, round_body, 0)
        pltpu.sync_copy(lc_v, sh_v.at[wid])

    plsc.subcore_barrier()

    @pl.when(on_core & (wid == 0))
    def _phase_b():
        pltpu.sync_copy(sh_v, mc_v)
        heads0 = plsc.bitcast(
            plsc.load_gather(mc_v, [lane, jnp.zeros((16,), jnp.int32)]),
            jnp.float32)
        ptr0 = jnp.zeros((16,), jnp.int32)

        def merge_body(r, carry):
            heads, ptr = carry
            m = _allmax(heads)
            l = _allmin(jnp.where(heads == m, lane, _NTILE))
            eq = lane == l
            pv = _shuf(ptr, l)
            iv = plsc.load_gather(mc_v, [l, pv + FINAL_K])
            plsc.store_scatter(fi_v, [_bc16(r)], iv, mask=lane0)
            nxt = jnp.minimum(pv + 1, FINAL_K - 1)
            newhead = plsc.bitcast(plsc.load_gather(mc_v, [l, nxt]),
                                   jnp.float32)
            heads = jnp.where(eq & (pv + 1 >= FINAL_K), _NEG,
                              jnp.where(eq, newhead, heads))
            ptr = jnp.where(eq, ptr + 1, ptr)
            return (heads, ptr)

        lax.fori_loop(0, FINAL_K, merge_body, (heads0, ptr0))
        pltpu.sync_copy(fi_v, sh_v.at[0])

    plsc.subcore_barrier()

    @pl.when(on_core & (wid < 8))
    def _phase_c():
        pltpu.sync_copy(sh_v.at[0, pl.ds(wid * 8, 8)], idx8_v)
        pltpu.async_copy(x_hbm.at[idx8_v], rows_v, sem).wait()
        pltpu.sync_copy(rows_v, out_hbm.at[pl.ds(wid * 8, 8)])


def _sc_topk_gather(z_flat, x_flat):
    mesh = plsc.VectorSubcoreMesh(core_axis_name="c", subcore_axis_name="s")
    f = functools.partial(
        pl.kernel,
        out_type=jax.ShapeDtypeStruct((FINAL_K, D), jnp.float32),
        mesh=mesh,
        compiler_params=pltpu.CompilerParams(needs_layout_passes=False),
        scratch_types=[
            pltpu.VMEM((_PERW,), jnp.float32),
            pltpu.VMEM((2 * FINAL_K,), jnp.int32),
            pltpu.VMEM_SHARED((_NTILE, 2 * FINAL_K), jnp.int32),
            pltpu.VMEM((_NTILE, 2 * FINAL_K), jnp.int32),
            pltpu.VMEM((2 * FINAL_K,), jnp.int32),
            pltpu.VMEM((8,), jnp.int32),
            pltpu.VMEM((8, D), jnp.float32),
            pltpu.SemaphoreType.DMA,
        ],
    )(_sc_topk_body)
    return f(z_flat, x_flat)


def _ln_ref(x, w=None, b=None, eps=1e-5):
    mu = jnp.mean(x, axis=-1, keepdims=True)
    var = jnp.mean((x - mu) ** 2, axis=-1, keepdims=True)
    y = (x - mu) / jnp.sqrt(var + eps)
    if w is not None:
        y = y * w + b
    return y


def kernel(text_features, projected_layer_features, W1_w, W1_b, Wc_w, Wc_b,
           Wi_w, Wi_b, Wf_w, Wf_b, bc, bi, bf, Wq, Wk, ln_w, ln_b):
    X = projected_layer_features

    text_global = _ln_ref(jnp.mean(text_features, axis=0))
    y = jnp.mean(X, axis=1)

    def step(c_prev, y_l):
        c_prev_norm = jax.nn.sigmoid(c_prev)
        combined = jnp.concatenate([c_prev_norm, y_l, text_global], axis=-1)
        s = jax.nn.relu(combined @ W1_w.T + W1_b)
        c_tilde = jnp.tanh(s @ Wc_w.T + Wc_b + bc)
        i = jax.nn.sigmoid(s @ Wi_w.T + Wi_b + bi)
        f = jax.nn.sigmoid(s @ Wf_w.T + Wf_b + bf)
        c_l = f * c_prev + i * c_tilde
        return c_l, c_l

    c0 = jnp.zeros((D,), dtype=text_features.dtype)
    _, contexts = jax.lax.scan(step, c0, y)
    q = _ln_ref(contexts[-2] @ Wq.T, ln_w, ln_b)

    wq = (ln_w * q)
    sb = jnp.stack([jnp.sum(wq), jnp.dot(ln_b, q)])

    z = pl.pallas_call(
        _score_body,
        grid=(L,),
        in_specs=[
            pl.BlockSpec((1, N, D), lambda l: (l, 0, 0)),
            pl.BlockSpec((D, D), lambda l: (0, 0)),
            pl.BlockSpec((D, 1), lambda l: (0, 0)),
            pl.BlockSpec(memory_space=pltpu.SMEM),
        ],
        out_specs=pl.BlockSpec((1, 1, N), lambda l: (l, 0, 0)),
        out_shape=jax.ShapeDtypeStruct((L, 1, N), jnp.float32),
    )(X, Wk, wq.reshape(D, 1), sb)

    return _sc_topk_gather(z.reshape(-1), X.reshape(-1, D))
